# Initial kernel scaffold; baseline (speedup 1.0000x reference)
import jax
import jax.numpy as jnp
from jax.experimental import pallas as pl


def kernel(x, Wr, br, We, be):
    raise NotImplementedError("write your pallas kernel here")


# speedup vs baseline: 1.7393x; 1.7393x over previous
import functools

import jax
import jax.numpy as jnp
from jax.experimental import pallas as pl
from jax.experimental.pallas import tpu as pltpu

_TOPK = 8


def _routing_body(T, E, x_ref, wrt_ref, br_ref, probs_ref, fp_ref, lbl_ref,
                  acc_p_ref, acc_s_ref):
  i = pl.program_id(0)
  nt = pl.num_programs(0)
  logits = jnp.dot(x_ref[...].astype(jnp.bfloat16),
                   wrt_ref[...].astype(jnp.bfloat16),
                   preferred_element_type=jnp.float32) + br_ref[...]
  iot = jax.lax.broadcasted_iota(jnp.int32, logits.shape, 1)
  avail = jnp.ones(logits.shape, dtype=jnp.bool_)
  sel = jnp.zeros(logits.shape, dtype=jnp.bool_)
  onehot1 = None
  for k in range(_TOPK):
    masked = jnp.where(avail, logits, -jnp.inf)
    am = jnp.argmax(masked, axis=1)
    oh = iot == am[:, None]
    sel = jnp.logical_or(sel, oh)
    avail = jnp.logical_and(avail, jnp.logical_not(oh))
    if k == 0:
      onehot1 = oh
  mx = jnp.max(logits, axis=1, keepdims=True)
  ex = jnp.where(sel, jnp.exp(logits - mx), 0.0)
  p = ex / jnp.sum(ex, axis=1, keepdims=True)
  probs_ref[...] = p

  @pl.when(i == 0)
  def _init():
    acc_p_ref[...] = jnp.zeros_like(acc_p_ref)
    acc_s_ref[...] = jnp.zeros_like(acc_s_ref)

  acc_p_ref[...] += jnp.sum(p, axis=0, keepdims=True)
  acc_s_ref[...] += jnp.sum(onehot1.astype(jnp.float32), axis=0, keepdims=True)

  @pl.when(i == nt - 1)
  def _fin():
    fp = acc_p_ref[...] / jnp.float32(T)
    fs = acc_s_ref[...] / jnp.float32(T)
    fp_ref[...] = fp
    lbl_ref[...] = jnp.full((1, 1), jnp.float32(E)) * jnp.sum(
        fs * fp, keepdims=True)


def _moe_body(p_ref, x_ref, we_ref, be_ref, out_ref):
  e = pl.program_id(2)
  p = p_ref[...]

  @pl.when(e == 0)
  def _init():
    out_ref[...] = jnp.dot(p, be_ref[...],
                           preferred_element_type=jnp.float32,
                           precision=jax.lax.Precision.HIGHEST)

  acc = jnp.dot(x_ref[...], we_ref[0],
                preferred_element_type=jnp.float32)
  iot = jax.lax.broadcasted_iota(jnp.int32, p.shape, 1)
  pcol = jnp.sum(jnp.where(iot == e, p, 0.0), axis=1, keepdims=True)
  out_ref[...] += acc * pcol


def kernel(x, Wr, br, We, be):
  T, H_IN = x.shape
  E = Wr.shape[0]
  H_OUT = We.shape[2]

  bt_r = min(1024, T)
  routing = pl.pallas_call(
      functools.partial(_routing_body, T, E),
      grid=(T // bt_r,),
      in_specs=[
          pl.BlockSpec((bt_r, H_IN), lambda i: (i, 0)),
          pl.BlockSpec((H_IN, E), lambda i: (0, 0)),
          pl.BlockSpec((1, E), lambda i: (0, 0)),
      ],
      out_specs=[
          pl.BlockSpec((bt_r, E), lambda i: (i, 0)),
          pl.BlockSpec((1, E), lambda i: (0, 0)),
          pl.BlockSpec((1, 1), lambda i: (0, 0)),
      ],
      out_shape=[
          jax.ShapeDtypeStruct((T, E), jnp.float32),
          jax.ShapeDtypeStruct((1, E), jnp.float32),
          jax.ShapeDtypeStruct((1, 1), jnp.float32),
      ],
      scratch_shapes=[
          pltpu.VMEM((1, E), jnp.float32),
          pltpu.VMEM((1, E), jnp.float32),
      ],
      compiler_params=pltpu.CompilerParams(
          dimension_semantics=("arbitrary",)),
  )
  probs, fp2, lbl2 = routing(x, Wr.T, br.reshape(1, E))

  bt = min(2048, T)
  bh = min(512, H_OUT)
  moe = pl.pallas_call(
      _moe_body,
      grid=(T // bt, H_OUT // bh, E),
      in_specs=[
          pl.BlockSpec((bt, E), lambda t, h, e: (t, 0)),
          pl.BlockSpec((bt, H_IN), lambda t, h, e: (t, 0)),
          pl.BlockSpec((1, H_IN, bh), lambda t, h, e: (e, 0, h)),
          pl.BlockSpec((E, bh), lambda t, h, e: (0, h)),
      ],
      out_specs=pl.BlockSpec((bt, bh), lambda t, h, e: (t, h)),
      out_shape=jax.ShapeDtypeStruct((T, H_OUT), jnp.float32),
      compiler_params=pltpu.CompilerParams(
          dimension_semantics=("parallel", "parallel", "arbitrary")),
  )
  out = moe(probs, x.astype(jnp.bfloat16), We.astype(jnp.bfloat16), be)

  return (out, fp2.reshape(E, 1), lbl2.reshape(()))


# speedup vs baseline: 1.9040x; 1.0947x over previous
import functools

import jax
import jax.numpy as jnp
from jax.experimental import pallas as pl
from jax.experimental.pallas import tpu as pltpu

_TOPK = 8


def _routing_body(T, E, x_ref, wrt_ref, br_ref, probs_ref, fp_ref, lbl_ref,
                  acc_p_ref, acc_s_ref):
  i = pl.program_id(0)
  nt = pl.num_programs(0)
  logits = jnp.dot(x_ref[...],
                   wrt_ref[...].astype(jnp.bfloat16),
                   preferred_element_type=jnp.float32) + br_ref[...]
  iot = jax.lax.broadcasted_iota(jnp.int32, logits.shape, 1)
  avail = jnp.ones(logits.shape, dtype=jnp.bool_)
  sel = jnp.zeros(logits.shape, dtype=jnp.bool_)
  onehot1 = None
  for k in range(_TOPK):
    masked = jnp.where(avail, logits, -jnp.inf)
    am = jnp.argmax(masked, axis=1)
    oh = iot == am[:, None]
    sel = jnp.logical_or(sel, oh)
    avail = jnp.logical_and(avail, jnp.logical_not(oh))
    if k == 0:
      onehot1 = oh
  mx = jnp.max(logits, axis=1, keepdims=True)
  ex = jnp.where(sel, jnp.exp(logits - mx), 0.0)
  p = ex / jnp.sum(ex, axis=1, keepdims=True)
  probs_ref[...] = p

  @pl.when(i == 0)
  def _init():
    acc_p_ref[...] = jnp.zeros_like(acc_p_ref)
    acc_s_ref[...] = jnp.zeros_like(acc_s_ref)

  acc_p_ref[...] += jnp.sum(p, axis=0, keepdims=True)
  acc_s_ref[...] += jnp.sum(onehot1.astype(jnp.float32), axis=0, keepdims=True)

  @pl.when(i == nt - 1)
  def _fin():
    fp = acc_p_ref[...] / jnp.float32(T)
    fs = acc_s_ref[...] / jnp.float32(T)
    fp_ref[...] = fp
    lbl_ref[...] = jnp.full((1, 1), jnp.float32(E)) * jnp.sum(
        fs * fp, keepdims=True)


def _moe_body(p_ref, x_ref, we_ref, be_ref, out_ref):
  e = pl.program_id(2)
  p = p_ref[...]

  @pl.when(e == 0)
  def _init():
    out_ref[...] = jnp.dot(p, be_ref[...],
                           preferred_element_type=jnp.float32,
                           precision=jax.lax.Precision.HIGHEST)

  acc = jnp.dot(x_ref[...], we_ref[0].astype(jnp.bfloat16),
                preferred_element_type=jnp.float32)
  iot = jax.lax.broadcasted_iota(jnp.int32, p.shape, 1)
  pcol = jnp.sum(jnp.where(iot == e, p, 0.0), axis=1, keepdims=True)
  out_ref[...] += acc * pcol


def kernel(x, Wr, br, We, be):
  T, H_IN = x.shape
  E = Wr.shape[0]
  H_OUT = We.shape[2]

  bt_r = min(1024, T)
  routing = pl.pallas_call(
      functools.partial(_routing_body, T, E),
      grid=(T // bt_r,),
      in_specs=[
          pl.BlockSpec((bt_r, H_IN), lambda i: (i, 0)),
          pl.BlockSpec((H_IN, E), lambda i: (0, 0)),
          pl.BlockSpec((1, E), lambda i: (0, 0)),
      ],
      out_specs=[
          pl.BlockSpec((bt_r, E), lambda i: (i, 0)),
          pl.BlockSpec((1, E), lambda i: (0, 0)),
          pl.BlockSpec((1, 1), lambda i: (0, 0)),
      ],
      out_shape=[
          jax.ShapeDtypeStruct((T, E), jnp.float32),
          jax.ShapeDtypeStruct((1, E), jnp.float32),
          jax.ShapeDtypeStruct((1, 1), jnp.float32),
      ],
      scratch_shapes=[
          pltpu.VMEM((1, E), jnp.float32),
          pltpu.VMEM((1, E), jnp.float32),
      ],
      compiler_params=pltpu.CompilerParams(
          dimension_semantics=("arbitrary",)),
  )
  xb = x.astype(jnp.bfloat16)
  probs, fp2, lbl2 = routing(xb, Wr.T, br.reshape(1, E))

  bt = min(2048, T)
  bh = min(512, H_OUT)
  moe = pl.pallas_call(
      _moe_body,
      grid=(T // bt, H_OUT // bh, E),
      in_specs=[
          pl.BlockSpec((bt, E), lambda t, h, e: (t, 0)),
          pl.BlockSpec((bt, H_IN), lambda t, h, e: (t, 0)),
          pl.BlockSpec((1, H_IN, bh), lambda t, h, e: (e, 0, h)),
          pl.BlockSpec((E, bh), lambda t, h, e: (0, h)),
      ],
      out_specs=pl.BlockSpec((bt, bh), lambda t, h, e: (t, h)),
      out_shape=jax.ShapeDtypeStruct((T, H_OUT), jnp.float32),
      compiler_params=pltpu.CompilerParams(
          dimension_semantics=("parallel", "parallel", "arbitrary")),
  )
  out = moe(probs, xb, We, be)

  return (out, fp2.reshape(E, 1), lbl2.reshape(()))


# speedup vs baseline: 2.0331x; 1.0678x over previous
import functools

import jax
import jax.numpy as jnp
from jax.experimental import pallas as pl
from jax.experimental.pallas import tpu as pltpu

_TOPK = 8


def _routing_body(T, E, x_ref, wrt_ref, br_ref, probs_ref, fp_ref, lbl_ref,
                  acc_p_ref, acc_s_ref):
  i = pl.program_id(0)
  nt = pl.num_programs(0)
  logits = jnp.dot(x_ref[...],
                   wrt_ref[...].astype(jnp.bfloat16),
                   preferred_element_type=jnp.float32) + br_ref[...]
  iot = jax.lax.broadcasted_iota(jnp.int32, logits.shape, 1)
  avail = jnp.ones(logits.shape, dtype=jnp.bool_)
  sel = jnp.zeros(logits.shape, dtype=jnp.bool_)
  onehot1 = None
  for k in range(_TOPK):
    masked = jnp.where(avail, logits, -jnp.inf)
    am = jnp.argmax(masked, axis=1)
    oh = iot == am[:, None]
    sel = jnp.logical_or(sel, oh)
    avail = jnp.logical_and(avail, jnp.logical_not(oh))
    if k == 0:
      onehot1 = oh
  mx = jnp.max(logits, axis=1, keepdims=True)
  ex = jnp.where(sel, jnp.exp(logits - mx), 0.0)
  p = ex / jnp.sum(ex, axis=1, keepdims=True)
  probs_ref[...] = p

  @pl.when(i == 0)
  def _init():
    acc_p_ref[...] = jnp.zeros_like(acc_p_ref)
    acc_s_ref[...] = jnp.zeros_like(acc_s_ref)

  acc_p_ref[...] += jnp.sum(p, axis=0, keepdims=True)
  acc_s_ref[...] += jnp.sum(onehot1.astype(jnp.float32), axis=0, keepdims=True)

  @pl.when(i == nt - 1)
  def _fin():
    fp = acc_p_ref[...] / jnp.float32(T)
    fs = acc_s_ref[...] / jnp.float32(T)
    fp_ref[...] = fp
    lbl_ref[...] = jnp.full((1, 1), jnp.float32(E)) * jnp.sum(
        fs * fp, keepdims=True)


def _moe_body(eb, p_ref, x_ref, we_ref, be_ref, out_ref):
  e4 = pl.program_id(2)
  p = p_ref[...]
  x = x_ref[...]
  iot = jax.lax.broadcasted_iota(jnp.int32, p.shape, 1)

  contrib = None
  for j in range(eb):
    acc = jnp.dot(x, we_ref[j].astype(jnp.bfloat16),
                  preferred_element_type=jnp.float32)
    pcol = jnp.sum(jnp.where(iot == e4 * eb + j, p, 0.0),
                   axis=1, keepdims=True)
    term = acc * pcol
    contrib = term if contrib is None else contrib + term

  @pl.when(e4 == 0)
  def _init():
    out_ref[...] = contrib + jnp.dot(
        p.astype(jnp.bfloat16), be_ref[...].astype(jnp.bfloat16),
        preferred_element_type=jnp.float32)

  @pl.when(e4 != 0)
  def _acc():
    out_ref[...] += contrib


def kernel(x, Wr, br, We, be):
  T, H_IN = x.shape
  E = Wr.shape[0]
  H_OUT = We.shape[2]

  bt_r = min(1024, T)
  routing = pl.pallas_call(
      functools.partial(_routing_body, T, E),
      grid=(T // bt_r,),
      in_specs=[
          pl.BlockSpec((bt_r, H_IN), lambda i: (i, 0)),
          pl.BlockSpec((H_IN, E), lambda i: (0, 0)),
          pl.BlockSpec((1, E), lambda i: (0, 0)),
      ],
      out_specs=[
          pl.BlockSpec((bt_r, E), lambda i: (i, 0)),
          pl.BlockSpec((1, E), lambda i: (0, 0)),
          pl.BlockSpec((1, 1), lambda i: (0, 0)),
      ],
      out_shape=[
          jax.ShapeDtypeStruct((T, E), jnp.float32),
          jax.ShapeDtypeStruct((1, E), jnp.float32),
          jax.ShapeDtypeStruct((1, 1), jnp.float32),
      ],
      scratch_shapes=[
          pltpu.VMEM((1, E), jnp.float32),
          pltpu.VMEM((1, E), jnp.float32),
      ],
      compiler_params=pltpu.CompilerParams(
          dimension_semantics=("arbitrary",)),
  )
  xb = x.astype(jnp.bfloat16)
  probs, fp2, lbl2 = routing(xb, Wr.T, br.reshape(1, E))

  bt = min(2048, T)
  bh = min(256, H_OUT)
  eb = min(4, E)
  moe = pl.pallas_call(
      functools.partial(_moe_body, eb),
      grid=(T // bt, H_OUT // bh, E // eb),
      in_specs=[
          pl.BlockSpec((bt, E), lambda t, h, e: (t, 0)),
          pl.BlockSpec((bt, H_IN), lambda t, h, e: (t, 0)),
          pl.BlockSpec((eb, H_IN, bh), lambda t, h, e: (e, 0, h)),
          pl.BlockSpec((E, bh), lambda t, h, e: (0, h)),
      ],
      out_specs=pl.BlockSpec((bt, bh), lambda t, h, e: (t, h)),
      out_shape=jax.ShapeDtypeStruct((T, H_OUT), jnp.float32),
      compiler_params=pltpu.CompilerParams(
          dimension_semantics=("parallel", "parallel", "arbitrary")),
  )
  out = moe(probs, xb, We, be)

  return (out, fp2.reshape(E, 1), lbl2.reshape(()))
